# Initial kernel scaffold; baseline (speedup 1.0000x reference)
import jax
import jax.numpy as jnp
from jax.experimental import pallas as pl


def kernel(inputs, discrete_shift, discrete_scale, discrete_idx):
    raise NotImplementedError("write your pallas kernel here")


# speedup vs baseline: 1.0853x; 1.0853x over previous
import functools

import jax
import jax.numpy as jnp
from jax import lax
from jax.experimental import pallas as pl
from jax.experimental.pallas import tpu as pltpu
from jax.experimental.pallas import tpu_sc as plsc

_B = 16384
_D = 128
_ND = 32
_NC = 2
_NS = 16
_NW = _NC * _NS
_RPW = _B // _NW

_ROTS = ((13, 15, 26, 6), (17, 29, 16, 24))
_KS = (0x0, 0x1, 0x1BD11BDB)


def _rotl(x, r):
    return lax.shift_left(x, jnp.uint32(r)) | lax.shift_right_logical(
        x, jnp.uint32(32 - r))


def _threefry_bits(c2):
    x0 = c2 ^ c2
    x1 = c2 + jnp.uint32(_KS[1])
    for i in range(5):
        for r in _ROTS[i % 2]:
            x0 = x0 + x1
            x1 = _rotl(x1, r)
            x1 = x0 ^ x1
        x0 = x0 + jnp.uint32(_KS[(i + 1) % 3])
        x1 = x1 + jnp.uint32((_KS[(i + 2) % 3] + i + 1) & 0xFFFFFFFF)
    return x0 ^ x1


def _sc_body(in_hbm, shift_hbm, scale_hbm, idx_hbm, out_hbm,
             buf, shift_v, scale_v, idx_v):
    c = lax.axis_index("c")
    s = lax.axis_index("s")
    wid = s * _NC + c
    row0 = wid * _RPW

    pltpu.sync_copy(shift_hbm, shift_v)
    pltpu.sync_copy(scale_hbm, scale_v)
    pltpu.sync_copy(idx_hbm, idx_v)
    pltpu.sync_copy(in_hbm.at[pl.ds(row0, _RPW)], buf)

    lane_u = lax.iota(jnp.uint32, 16)
    lane_i = lax.iota(jnp.int32, 16)
    cols = [idx_v[pl.ds(0, 16)], idx_v[pl.ds(16, 16)]]
    shs = [shift_v[pl.ds(0, 16)], shift_v[pl.ds(16, 16)]]
    scs = [scale_v[pl.ds(0, 16)], scale_v[pl.ds(16, 16)]]
    one = jnp.float32(1.0)
    invs = [one / scs[0], one / scs[1]]
    base0 = lax.convert_element_type(row0 * _ND, jnp.uint32)

    def row_step(lr, carry):
        row_vec = lane_i * 0 + lr
        base = base0 + lax.convert_element_type(lr * _ND, jnp.uint32)
        for h in range(2):
            c2 = lane_u + (base + jnp.uint32(16 * h))
            bits = _threefry_bits(c2)
            mant = lax.convert_element_type(
                lax.shift_right_logical(bits, jnp.uint32(9)), jnp.int32)
            n = lax.convert_element_type(mant, jnp.float32) * jnp.float32(
                1.0 / 8388608.0) - jnp.float32(0.5)
            x = plsc.load_gather(buf, [row_vec, cols[h]])
            d = x * scs[h] + shs[h]
            new = (d + n - shs[h]) * invs[h]
            plsc.store_scatter(buf, [row_vec, cols[h]], new)
        return carry

    lax.fori_loop(0, _RPW, row_step, 0)
    pltpu.sync_copy(buf, out_hbm.at[pl.ds(row0, _RPW)])


@functools.lru_cache(maxsize=1)
def _sc_call():
    return pl.kernel(
        _sc_body,
        out_type=jax.ShapeDtypeStruct((_B, _D), jnp.float32),
        mesh=plsc.VectorSubcoreMesh(core_axis_name="c", subcore_axis_name="s",
                                    num_cores=_NC, num_subcores=_NS),
        compiler_params=pltpu.CompilerParams(needs_layout_passes=False),
        scratch_types=[
            pltpu.VMEM((_RPW, _D), jnp.float32),
            pltpu.VMEM((_ND,), jnp.float32),
            pltpu.VMEM((_ND,), jnp.float32),
            pltpu.VMEM((_ND,), jnp.int32),
        ],
    )


def kernel(inputs, discrete_shift, discrete_scale, discrete_idx):
    return _sc_call()(inputs, discrete_shift, discrete_scale, discrete_idx)
